# Initial kernel scaffold; baseline (speedup 1.0000x reference)
import jax
import jax.numpy as jnp
from jax.experimental import pallas as pl


def kernel(tokens, weight, pos_table, gamma, beta):
    raise NotImplementedError("write your pallas kernel here")


# speedup vs baseline: 1.2664x; 1.2664x over previous
import functools
import math

import jax
import jax.numpy as jnp
from jax import lax
from jax.experimental import pallas as pl
from jax.experimental.pallas import tpu as pltpu
from jax.experimental.pallas import tpu_sc as plsc

D = 1024
B = 32
SEQ = 1024
OFFSET = 2
EPS = 1e-5

NW = 32
ROWS_PER_W = (B * SEQ) // NW
CHUNK = 32
NCHUNK = ROWS_PER_W // CHUNK


def _sc_gather(tokens_t, weight):
    mesh = plsc.VectorSubcoreMesh(core_axis_name="c", subcore_axis_name="s")

    @functools.partial(
        pl.kernel,
        out_type=jax.ShapeDtypeStruct((B * SEQ, D), jnp.float32),
        mesh=mesh,
        scratch_types=[
            pltpu.VMEM((ROWS_PER_W,), jnp.int32),
            pltpu.VMEM((CHUNK, D), jnp.float32),
            pltpu.VMEM((CHUNK, D), jnp.float32),
            pltpu.SemaphoreType.DMA,
            pltpu.SemaphoreType.DMA,
        ],
    )
    def gather_kernel(tok_hbm, w_hbm, out_hbm, tok_v, buf_a, buf_b, sem_a, sem_b):
        wid = lax.axis_index("c") * 16 + lax.axis_index("s")
        base = wid * ROWS_PER_W
        pltpu.sync_copy(tok_hbm.at[pl.ds(base, ROWS_PER_W)], tok_v)
        bufs = (buf_a, buf_b)
        sems = (sem_a, sem_b)
        descs = [None, None]

        def start(c):
            p = c % 2
            idx = tok_v.at[pl.ds(c * CHUNK, CHUNK)]
            descs[p] = pltpu.async_copy(w_hbm.at[idx], bufs[p], sems[p])

        start(0)
        for c in range(NCHUNK):
            p = c % 2
            if c + 1 < NCHUNK:
                start(c + 1)
            descs[p].wait()
            pltpu.sync_copy(bufs[p], out_hbm.at[pl.ds(base + c * CHUNK, CHUNK)])

    return gather_kernel(tokens_t, weight)


SBLK = 8


def _tc_ln(embed_flat, pos_sl, gamma, beta):
    def body(emb_ref, pos_ref, g_ref, b_ref, out_ref):
        x = emb_ref[...].reshape(SBLK, B, D) * math.sqrt(D)
        x = x + pos_ref[...][:, None, :]
        mean = jnp.mean(x, axis=-1, keepdims=True)
        var = jnp.mean((x - mean) ** 2, axis=-1, keepdims=True)
        y = (x - mean) * lax.rsqrt(var + EPS) * g_ref[...] + b_ref[...]
        out_ref[...] = y.reshape(SBLK * B, D)

    return pl.pallas_call(
        body,
        grid=(SEQ // SBLK,),
        in_specs=[
            pl.BlockSpec((SBLK * B, D), lambda i: (i, 0)),
            pl.BlockSpec((SBLK, D), lambda i: (i, 0)),
            pl.BlockSpec((1, D), lambda i: (0, 0)),
            pl.BlockSpec((1, D), lambda i: (0, 0)),
        ],
        out_specs=pl.BlockSpec((SBLK * B, D), lambda i: (i, 0)),
        out_shape=jax.ShapeDtypeStruct((SEQ * B, D), jnp.float32),
    )(embed_flat, pos_sl, gamma.reshape(1, D), beta.reshape(1, D))


def kernel(tokens, weight, pos_table, gamma, beta):
    tokens_t = tokens.T.reshape(-1)
    embed_flat = _sc_gather(tokens_t, weight)
    pos_sl = lax.slice_in_dim(pos_table, OFFSET, OFFSET + SEQ, axis=0)
    out = _tc_ln(embed_flat, pos_sl, gamma, beta)
    return out.reshape(SEQ, B, D)


# speedup vs baseline: 1.5935x; 1.2583x over previous
import functools
import math

import jax
import jax.numpy as jnp
from jax import lax
from jax.experimental import pallas as pl
from jax.experimental.pallas import tpu as pltpu
from jax.experimental.pallas import tpu_sc as plsc

D = 1024
B = 32
SEQ = 1024
OFFSET = 2
EPS = 1e-5

NW = 32
ROWS_PER_W = (B * SEQ) // NW
CHUNK = 32
NCHUNK = ROWS_PER_W // CHUNK


def _sc_gather(tokens_t, weight):
    mesh = plsc.VectorSubcoreMesh(core_axis_name="c", subcore_axis_name="s")

    @functools.partial(
        pl.kernel,
        out_type=jax.ShapeDtypeStruct((B * SEQ, D), jnp.float32),
        mesh=mesh,
        scratch_types=[
            pltpu.VMEM((ROWS_PER_W,), jnp.int32),
            pltpu.VMEM((CHUNK, D), jnp.float32),
            pltpu.VMEM((CHUNK, D), jnp.float32),
            pltpu.SemaphoreType.DMA,
            pltpu.SemaphoreType.DMA,
        ],
    )
    def gather_kernel(tok_hbm, w_hbm, out_hbm, tok_v, buf_a, buf_b, sem_a, sem_b):
        wid = lax.axis_index("c") * 16 + lax.axis_index("s")
        base = wid * ROWS_PER_W
        pltpu.sync_copy(tok_hbm.at[pl.ds(base, ROWS_PER_W)], tok_v)
        bufs = (buf_a, buf_b)
        sems = (sem_a, sem_b)
        descs = [None, None]

        def start(c):
            p = c % 2
            idx = tok_v.at[pl.ds(c * CHUNK, CHUNK)]
            descs[p] = pltpu.async_copy(w_hbm.at[idx], bufs[p], sems[p])

        start(0)
        for c in range(NCHUNK):
            p = c % 2
            if c + 1 < NCHUNK:
                start(c + 1)
            descs[p].wait()
            pltpu.sync_copy(bufs[p], out_hbm.at[pl.ds(base + c * CHUNK, CHUNK)])

    return gather_kernel(tokens_t, weight)


SBLK = 32


def _tc_ln(embed_flat, pos_sl, gamma, beta):
    def body(emb_ref, pos_ref, g_ref, b_ref, out_ref):
        x = emb_ref[...].reshape(SBLK, B, D) * math.sqrt(D)
        x = x + pos_ref[...][:, None, :]
        mean = jnp.mean(x, axis=-1, keepdims=True)
        var = jnp.mean((x - mean) ** 2, axis=-1, keepdims=True)
        y = (x - mean) * lax.rsqrt(var + EPS) * g_ref[...] + b_ref[...]
        out_ref[...] = y.reshape(SBLK * B, D)

    return pl.pallas_call(
        body,
        grid=(SEQ // SBLK,),
        in_specs=[
            pl.BlockSpec((SBLK * B, D), lambda i: (i, 0)),
            pl.BlockSpec((SBLK, D), lambda i: (i, 0)),
            pl.BlockSpec((1, D), lambda i: (0, 0)),
            pl.BlockSpec((1, D), lambda i: (0, 0)),
        ],
        out_specs=pl.BlockSpec((SBLK * B, D), lambda i: (i, 0)),
        out_shape=jax.ShapeDtypeStruct((SEQ * B, D), jnp.float32),
    )(embed_flat, pos_sl, gamma.reshape(1, D), beta.reshape(1, D))


def kernel(tokens, weight, pos_table, gamma, beta):
    tokens_t = tokens.T.reshape(-1)
    embed_flat = _sc_gather(tokens_t, weight)
    pos_sl = lax.slice_in_dim(pos_table, OFFSET, OFFSET + SEQ, axis=0)
    out = _tc_ln(embed_flat, pos_sl, gamma, beta)
    return out.reshape(SEQ, B, D)


# speedup vs baseline: 1.6196x; 1.0164x over previous
import functools
import math

import jax
import jax.numpy as jnp
from jax import lax
from jax.experimental import pallas as pl
from jax.experimental.pallas import tpu as pltpu
from jax.experimental.pallas import tpu_sc as plsc

D = 1024
B = 32
SEQ = 1024
OFFSET = 2
EPS = 1e-5

NW = 32
ROWS_PER_W = (B * SEQ) // NW
CHUNK = 32
NCHUNK = ROWS_PER_W // CHUNK


def _sc_gather(tokens_t, weight):
    mesh = plsc.VectorSubcoreMesh(core_axis_name="c", subcore_axis_name="s")

    @functools.partial(
        pl.kernel,
        out_type=jax.ShapeDtypeStruct((B * SEQ, D), jnp.float32),
        mesh=mesh,
        scratch_types=[
            pltpu.VMEM((ROWS_PER_W,), jnp.int32),
            pltpu.VMEM((CHUNK, D), jnp.float32),
            pltpu.VMEM((CHUNK, D), jnp.float32),
            pltpu.SemaphoreType.DMA,
            pltpu.SemaphoreType.DMA,
        ],
    )
    def gather_kernel(tok_hbm, w_hbm, out_hbm, tok_v, buf_a, buf_b, sem_a, sem_b):
        wid = lax.axis_index("c") * 16 + lax.axis_index("s")
        base = wid * ROWS_PER_W
        pltpu.sync_copy(tok_hbm.at[pl.ds(base, ROWS_PER_W)], tok_v)
        bufs = (buf_a, buf_b)
        sems = (sem_a, sem_b)
        descs = [None, None]

        def start(c):
            p = c % 2
            idx = tok_v.at[pl.ds(c * CHUNK, CHUNK)]
            descs[p] = pltpu.async_copy(w_hbm.at[idx], bufs[p], sems[p])

        start(0)
        for c in range(NCHUNK):
            p = c % 2
            if c + 1 < NCHUNK:
                start(c + 1)
            descs[p].wait()
            pltpu.sync_copy(bufs[p], out_hbm.at[pl.ds(base + c * CHUNK, CHUNK)])

    return gather_kernel(tokens_t, weight)


SBLK = 64


def _tc_ln(embed_flat, pos_sl, gamma, beta):
    def body(emb_ref, pos_ref, g_ref, b_ref, out_ref):
        x = emb_ref[...].reshape(SBLK, B, D) * math.sqrt(D)
        x = x + pos_ref[...][:, None, :]
        mean = jnp.mean(x, axis=-1, keepdims=True)
        var = jnp.mean((x - mean) ** 2, axis=-1, keepdims=True)
        y = (x - mean) * lax.rsqrt(var + EPS) * g_ref[...] + b_ref[...]
        out_ref[...] = y.reshape(SBLK * B, D)

    return pl.pallas_call(
        body,
        grid=(SEQ // SBLK,),
        in_specs=[
            pl.BlockSpec((SBLK * B, D), lambda i: (i, 0)),
            pl.BlockSpec((SBLK, D), lambda i: (i, 0)),
            pl.BlockSpec((1, D), lambda i: (0, 0)),
            pl.BlockSpec((1, D), lambda i: (0, 0)),
        ],
        out_specs=pl.BlockSpec((SBLK * B, D), lambda i: (i, 0)),
        out_shape=jax.ShapeDtypeStruct((SEQ * B, D), jnp.float32),
    )(embed_flat, pos_sl, gamma.reshape(1, D), beta.reshape(1, D))


def kernel(tokens, weight, pos_table, gamma, beta):
    tokens_t = tokens.T.reshape(-1)
    embed_flat = _sc_gather(tokens_t, weight)
    pos_sl = lax.slice_in_dim(pos_table, OFFSET, OFFSET + SEQ, axis=0)
    out = _tc_ln(embed_flat, pos_sl, gamma, beta)
    return out.reshape(SEQ, B, D)
